# Initial kernel scaffold; baseline (speedup 1.0000x reference)
import jax
import jax.numpy as jnp
from jax.experimental import pallas as pl


def kernel(weights, types, segment_ids):
    raise NotImplementedError("write your pallas kernel here")


# speedup vs baseline: 176.7827x; 176.7827x over previous
import functools

import jax
import jax.numpy as jnp
from jax import lax
from jax.experimental import pallas as pl
from jax.experimental.pallas import tpu as pltpu
from jax.experimental.pallas import tpu_sc as plsc

N_ATOMS = 1048576
N_TYPES = 100
N_SEG = 16384
NC = 2
NS = 16
NW = NC * NS
L = 16
CHUNK = N_ATOMS // NW
VECS = CHUNK // L
W_PAD = 128

_mesh = plsc.VectorSubcoreMesh(
    core_axis_name="c", subcore_axis_name="s", num_cores=NC, num_subcores=NS
)


@functools.partial(
    pl.kernel,
    out_type=jax.ShapeDtypeStruct((NW, N_SEG), jnp.float32),
    mesh=_mesh,
    compiler_params=pltpu.CompilerParams(needs_layout_passes=False),
    scratch_types=[
        pltpu.VMEM((W_PAD,), jnp.float32),
        pltpu.VMEM((CHUNK,), jnp.int32),
        pltpu.VMEM((CHUNK + L,), jnp.int32),
        pltpu.VMEM((N_SEG,), jnp.float32),
    ],
)
def _sc_segsum(w_hbm, t_hbm, s_hbm, out_hbm, w_v, t_v, s_v, acc_v):
    wid = lax.axis_index("s") * NC + lax.axis_index("c")
    base = wid * CHUNK

    pltpu.sync_copy(w_hbm, w_v)
    pltpu.sync_copy(t_hbm.at[pl.ds(base, CHUNK)], t_v)
    pltpu.sync_copy(s_hbm.at[pl.ds(base, CHUNK)], s_v.at[pl.ds(0, CHUNK)])

    zero = jnp.zeros((L,), jnp.float32)

    def zero_body(i, carry):
        acc_v[pl.ds(i * L, L)] = zero
        return carry

    lax.fori_loop(0, N_SEG // L, zero_body, 0)

    lane = lax.iota(jnp.int32, L)
    is_last = lane == (L - 1)
    not_last = lane < (L - 1)

    def body(i, carry):
        k = i * L
        t = t_v[pl.ds(k, L)]
        seg = s_v[pl.ds(k, L)]
        segn = s_v[pl.ds(k + 1, L)]
        w = plsc.load_gather(w_v, [t])
        c = plsc.cumsum(w)
        mb = (seg != segn) & not_last
        mp = mb | is_last
        plsc.addupdate_scatter(acc_v, [seg], c, mask=mp)
        segn_safe = jnp.where(mb, segn, 0)
        plsc.addupdate_scatter(acc_v, [segn_safe], -c, mask=mb)
        return carry

    lax.fori_loop(0, VECS, body, 0)

    pltpu.sync_copy(acc_v, out_hbm.at[wid])


def _reduce_body(x_ref, o_ref):
    o_ref[...] = jnp.sum(x_ref[...], axis=0, keepdims=True)


def kernel(weights, types, segment_ids):
    w_flat = jnp.zeros((W_PAD,), jnp.float32).at[:N_TYPES].set(
        weights.reshape(-1).astype(jnp.float32)
    )
    partials = _sc_segsum(
        w_flat, types.astype(jnp.int32), segment_ids.astype(jnp.int32)
    )
    out = pl.pallas_call(
        _reduce_body,
        out_shape=jax.ShapeDtypeStruct((1, N_SEG), jnp.float32),
    )(partials)
    return out.reshape(N_SEG, 1)


# speedup vs baseline: 323.3860x; 1.8293x over previous
import functools

import jax
import jax.numpy as jnp
from jax import lax
from jax.experimental import pallas as pl
from jax.experimental.pallas import tpu as pltpu
from jax.experimental.pallas import tpu_sc as plsc

N_ATOMS = 1048576
N_TYPES = 100
N_SEG = 16384
NC = 2
NS = 16
NW = NC * NS
L = 16
CHUNK = N_ATOMS // NW
VECS = CHUNK // L
W_PAD = 128

_mesh = plsc.VectorSubcoreMesh(
    core_axis_name="c", subcore_axis_name="s", num_cores=NC, num_subcores=NS
)


@functools.partial(
    pl.kernel,
    out_type=jax.ShapeDtypeStruct((NW, N_SEG), jnp.float32),
    mesh=_mesh,
    compiler_params=pltpu.CompilerParams(needs_layout_passes=False),
    scratch_types=[
        pltpu.VMEM((W_PAD,), jnp.float32),
        pltpu.VMEM((CHUNK,), jnp.int32),
        pltpu.VMEM((CHUNK + L,), jnp.int32),
        pltpu.VMEM((N_SEG,), jnp.float32),
    ],
)
def _sc_segsum(w_hbm, t_hbm, s_hbm, out_hbm, w_v, t_v, s_v, acc_v):
    wid = lax.axis_index("s") * NC + lax.axis_index("c")
    base = wid * CHUNK

    pltpu.sync_copy(w_hbm, w_v)
    pltpu.sync_copy(t_hbm.at[pl.ds(base, CHUNK)], t_v)
    pltpu.sync_copy(s_hbm.at[pl.ds(base, CHUNK)], s_v.at[pl.ds(0, CHUNK)])

    zero = jnp.zeros((L,), jnp.float32)

    @plsc.parallel_loop(0, N_SEG, step=L, unroll=8)
    def _(k):
        acc_v[pl.ds(k, L)] = zero

    lane = lax.iota(jnp.int32, L)
    is_last = lane == (L - 1)
    not_last = lane < (L - 1)

    @plsc.parallel_loop(0, CHUNK, step=L, unroll=8)
    def _(k):
        t = t_v[pl.ds(k, L)]
        seg = s_v[pl.ds(k, L)]
        segn = s_v[pl.ds(k + 1, L)]
        w = plsc.load_gather(w_v, [t])
        c = plsc.cumsum(w)
        mb = (seg != segn) & not_last
        mp = mb | is_last
        plsc.addupdate_scatter(acc_v, [seg], c, mask=mp)
        segn_safe = jnp.where(mb, segn, 0)
        plsc.addupdate_scatter(acc_v, [segn_safe], -c, mask=mb)

    pltpu.sync_copy(acc_v, out_hbm.at[wid])


def _reduce_body(x_ref, o_ref):
    o_ref[...] = jnp.sum(x_ref[...], axis=0, keepdims=True)


def kernel(weights, types, segment_ids):
    w_flat = jnp.zeros((W_PAD,), jnp.float32).at[:N_TYPES].set(
        weights.reshape(-1).astype(jnp.float32)
    )
    partials = _sc_segsum(
        w_flat, types.astype(jnp.int32), segment_ids.astype(jnp.int32)
    )
    out = pl.pallas_call(
        _reduce_body,
        out_shape=jax.ShapeDtypeStruct((1, N_SEG), jnp.float32),
    )(partials)
    return out.reshape(N_SEG, 1)


# speedup vs baseline: 323.3932x; 1.0000x over previous
import functools

import jax
import jax.numpy as jnp
from jax import lax
from jax.experimental import pallas as pl
from jax.experimental.pallas import tpu as pltpu
from jax.experimental.pallas import tpu_sc as plsc

N_ATOMS = 1048576
N_TYPES = 100
N_SEG = 16384
NC = 2
NS = 16
NW = NC * NS
L = 16
CHUNK = N_ATOMS // NW
VECS = CHUNK // L
W_PAD = 128

_mesh = plsc.VectorSubcoreMesh(
    core_axis_name="c", subcore_axis_name="s", num_cores=NC, num_subcores=NS
)


@functools.partial(
    pl.kernel,
    out_type=jax.ShapeDtypeStruct((NW, N_SEG), jnp.float32),
    mesh=_mesh,
    compiler_params=pltpu.CompilerParams(needs_layout_passes=False),
    scratch_types=[
        pltpu.VMEM((W_PAD,), jnp.float32),
        pltpu.VMEM((CHUNK,), jnp.int32),
        pltpu.VMEM((CHUNK + L,), jnp.int32),
        pltpu.VMEM((N_SEG,), jnp.float32),
    ],
)
def _sc_segsum(w_hbm, t_hbm, s_hbm, out_hbm, w_v, t_v, s_v, acc_v):
    wid = lax.axis_index("s") * NC + lax.axis_index("c")
    base = wid * CHUNK

    pltpu.sync_copy(w_hbm, w_v.at[pl.ds(0, N_TYPES)])
    pltpu.sync_copy(t_hbm.at[pl.ds(base, CHUNK)], t_v)
    pltpu.sync_copy(s_hbm.at[pl.ds(base, CHUNK)], s_v.at[pl.ds(0, CHUNK)])

    zero = jnp.zeros((L,), jnp.float32)

    @plsc.parallel_loop(0, N_SEG, step=L, unroll=8)
    def _(k):
        acc_v[pl.ds(k, L)] = zero

    lane = lax.iota(jnp.int32, L)
    is_last = lane == (L - 1)
    not_last = lane < (L - 1)
    @plsc.parallel_loop(0, CHUNK, step=L, unroll=16)
    def _(k):
        t = t_v[pl.ds(k, L)]
        seg = s_v[pl.ds(k, L)]
        segn = s_v[pl.ds(k + 1, L)]
        w = plsc.load_gather(w_v, [t])
        c = plsc.cumsum(w)
        mb = (seg != segn) & not_last
        mp = mb | is_last
        plsc.addupdate_scatter(acc_v, [seg], c, mask=mp)
        segn_safe = jnp.where(mb, segn, 0)
        plsc.addupdate_scatter(acc_v, [segn_safe], -c, mask=mb)

    pltpu.sync_copy(acc_v, out_hbm.at[wid])


def _reduce_body(x_ref, o_ref):
    o_ref[...] = jnp.sum(x_ref[...], axis=0, keepdims=True)


def kernel(weights, types, segment_ids):
    partials = _sc_segsum(
        weights.reshape(-1).astype(jnp.float32),
        types.astype(jnp.int32),
        segment_ids.astype(jnp.int32),
    )
    out = pl.pallas_call(
        _reduce_body,
        out_shape=jax.ShapeDtypeStruct((1, N_SEG), jnp.float32),
    )(partials)
    return out.reshape(N_SEG, 1)


# speedup vs baseline: 334.4875x; 1.0343x over previous
import functools

import jax
import jax.numpy as jnp
from jax import lax
from jax.experimental import pallas as pl
from jax.experimental.pallas import tpu as pltpu
from jax.experimental.pallas import tpu_sc as plsc

N_ATOMS = 1048576
N_TYPES = 100
N_SEG = 16384
NC = 2
NS = 16
NW = NC * NS
L = 16
CHUNK = N_ATOMS // NW
VECS = CHUNK // L
W_PAD = 128
NQ = 4
QC = CHUNK // NQ

_mesh = plsc.VectorSubcoreMesh(
    core_axis_name="c", subcore_axis_name="s", num_cores=NC, num_subcores=NS
)


@functools.partial(
    pl.kernel,
    out_type=jax.ShapeDtypeStruct((NW, N_SEG), jnp.float32),
    mesh=_mesh,
    compiler_params=pltpu.CompilerParams(needs_layout_passes=False),
    scratch_types=[
        pltpu.VMEM((W_PAD,), jnp.float32),
        pltpu.VMEM((CHUNK,), jnp.int32),
        pltpu.VMEM((CHUNK + L,), jnp.int32),
        pltpu.VMEM((N_SEG,), jnp.float32),
        pltpu.SemaphoreType.DMA,
        [pltpu.SemaphoreType.DMA] * NQ,
        [pltpu.SemaphoreType.DMA] * NQ,
    ],
)
def _sc_segsum(w_hbm, t_hbm, s_hbm, out_hbm, w_v, t_v, s_v, acc_v,
               w_sem, t_sems, s_sems):
    wid = lax.axis_index("s") * NC + lax.axis_index("c")
    base = wid * CHUNK

    w_cp = pltpu.async_copy(w_hbm, w_v.at[pl.ds(0, N_TYPES)], w_sem)
    t_cps = []
    s_cps = []
    for q in range(NQ):
        t_cps.append(pltpu.async_copy(
            t_hbm.at[pl.ds(base + q * QC, QC)],
            t_v.at[pl.ds(q * QC, QC)], t_sems[q]))
        s_cps.append(pltpu.async_copy(
            s_hbm.at[pl.ds(base + q * QC, QC)],
            s_v.at[pl.ds(q * QC, QC)], s_sems[q]))

    zero = jnp.zeros((L,), jnp.float32)

    @plsc.parallel_loop(0, N_SEG, step=L, unroll=8)
    def _(k):
        acc_v[pl.ds(k, L)] = zero

    lane = lax.iota(jnp.int32, L)
    is_last = lane == (L - 1)
    not_last = lane < (L - 1)

    w_cp.wait()
    for q in range(NQ):
        t_cps[q].wait()
        s_cps[q].wait()

        @plsc.parallel_loop(q * QC, (q + 1) * QC, step=L, unroll=16)
        def _(k):
            t = t_v[pl.ds(k, L)]
            seg = s_v[pl.ds(k, L)]
            segn = s_v[pl.ds(k + 1, L)]
            w = plsc.load_gather(w_v, [t])
            c = plsc.cumsum(w)
            mb = (seg != segn) & not_last
            mp = mb | is_last
            plsc.addupdate_scatter(acc_v, [seg], c, mask=mp)
            segn_safe = jnp.where(mb, segn, 0)
            plsc.addupdate_scatter(acc_v, [segn_safe], -c, mask=mb)

    pltpu.sync_copy(acc_v, out_hbm.at[wid])


def _reduce_body(x_ref, o_ref):
    o_ref[...] = jnp.sum(x_ref[...], axis=0, keepdims=True)


def kernel(weights, types, segment_ids):
    partials = _sc_segsum(
        weights.reshape(-1).astype(jnp.float32),
        types.astype(jnp.int32),
        segment_ids.astype(jnp.int32),
    )
    out = pl.pallas_call(
        _reduce_body,
        out_shape=jax.ShapeDtypeStruct((1, N_SEG), jnp.float32),
    )(partials)
    return out.reshape(N_SEG, 1)
